# baseline (speedup 1.0000x reference)
import jax
import jax.numpy as jnp
from jax import lax
from jax.experimental import pallas as pl
from jax.experimental.pallas import tpu as pltpu


def _se_body(x_ref, w1_ref, b1_ref, w2_ref, b2_ref, o_ref, *, inv_hw):
    x = x_ref[...]
    s = jnp.sum(x.astype(jnp.float32), axis=-1) * inv_hw
    h = lax.dot_general(s, w1_ref[...], (((1,), (1,)), ((), ())),
                        preferred_element_type=jnp.float32)
    h = jnp.maximum(h + b1_ref[...], 0.0)
    y = lax.dot_general(h, w2_ref[...], (((1,), (1,)), ((), ())),
                        preferred_element_type=jnp.float32)
    y = jnp.clip(y + b2_ref[...], 0.0, 1.0)
    o_ref[...] = x * y[:, :, None].astype(x.dtype)


def kernel(x, w1, b1, w2, b2):
    B, C, H, W = x.shape
    HW = H * W
    hidden = w1.shape[0]
    x_flat = x.reshape(B, C, HW)
    b1r = b1.reshape(1, hidden)
    b2r = b2.reshape(1, C)

    row_bytes = C * HW * x.dtype.itemsize
    BT = max(1, min(B, (55 << 20) // (3 * row_bytes)))
    grid = (pl.cdiv(B, BT),)

    body = lambda *refs: _se_body(*refs, inv_hw=1.0 / float(HW))
    out_flat = pl.pallas_call(
        body,
        out_shape=jax.ShapeDtypeStruct((B, C, HW), x.dtype),
        grid=grid,
        in_specs=[
            pl.BlockSpec((BT, C, HW), lambda b: (b, 0, 0)),
            pl.BlockSpec(w1.shape, lambda b: (0, 0)),
            pl.BlockSpec(b1r.shape, lambda b: (0, 0)),
            pl.BlockSpec(w2.shape, lambda b: (0, 0)),
            pl.BlockSpec(b2r.shape, lambda b: (0, 0)),
        ],
        out_specs=pl.BlockSpec((BT, C, HW), lambda b: (b, 0, 0),
                               pipeline_mode=pl.Buffered(buffer_count=1)),
        compiler_params=pltpu.CompilerParams(
            dimension_semantics=("parallel",),
            vmem_limit_bytes=96 << 20,
        ),
        cost_estimate=pl.CostEstimate(
            flops=int(4 * B * C * hidden + 2 * B * C * HW),
            transcendentals=0,
            bytes_accessed=int(2 * B * C * HW * x.dtype.itemsize),
        ),
    )(x_flat, w1, b1r, w2, b2r)
    return out_flat.reshape(B, C, H, W)


# speedup vs baseline: 1.0094x; 1.0094x over previous
import jax
import jax.numpy as jnp
from jax import lax
from jax.experimental import pallas as pl
from jax.experimental.pallas import tpu as pltpu


def _se_body(x_ref, w1_ref, b1_ref, w2_ref, b2_ref, o_ref, *, inv_hw):
    x = x_ref[...]
    s = jnp.sum(x.astype(jnp.float32), axis=-1) * inv_hw
    h = lax.dot_general(s, w1_ref[...], (((1,), (1,)), ((), ())),
                        preferred_element_type=jnp.float32)
    h = jnp.maximum(h + b1_ref[...], 0.0)
    y = lax.dot_general(h, w2_ref[...], (((1,), (1,)), ((), ())),
                        preferred_element_type=jnp.float32)
    y = jnp.clip(y + b2_ref[...], 0.0, 1.0)
    o_ref[...] = x * y[:, :, None].astype(x.dtype)


def kernel(x, w1, b1, w2, b2):
    B, C, H, W = x.shape
    HW = H * W
    hidden = w1.shape[0]
    x_flat = x.reshape(B, C, HW)
    b1r = b1.reshape(1, hidden)
    b2r = b2.reshape(1, C)

    row_bytes = C * HW * x.dtype.itemsize
    BT = max(1, min(B, (62 << 20) // (4 * row_bytes)))
    grid = (pl.cdiv(B, BT),)

    body = lambda *refs: _se_body(*refs, inv_hw=1.0 / float(HW))
    out_flat = pl.pallas_call(
        body,
        out_shape=jax.ShapeDtypeStruct((B, C, HW), x.dtype),
        grid=grid,
        in_specs=[
            pl.BlockSpec((BT, C, HW), lambda b: (b, 0, 0)),
            pl.BlockSpec(w1.shape, lambda b: (0, 0)),
            pl.BlockSpec(b1r.shape, lambda b: (0, 0)),
            pl.BlockSpec(w2.shape, lambda b: (0, 0)),
            pl.BlockSpec(b2r.shape, lambda b: (0, 0)),
        ],
        out_specs=pl.BlockSpec((BT, C, HW), lambda b: (b, 0, 0)),
        compiler_params=pltpu.CompilerParams(
            dimension_semantics=("parallel",),
            vmem_limit_bytes=64 << 20,
        ),
        cost_estimate=pl.CostEstimate(
            flops=int(4 * B * C * hidden + 2 * B * C * HW),
            transcendentals=0,
            bytes_accessed=int(2 * B * C * HW * x.dtype.itemsize),
        ),
    )(x_flat, w1, b1r, w2, b2r)
    return out_flat.reshape(B, C, H, W)
